# baseline (speedup 1.0000x reference)
import functools

import jax
import jax.numpy as jnp
from jax import lax
from jax.experimental import pallas as pl
from jax.experimental.pallas import tpu as pltpu
from jax.experimental.pallas import tpu_sc as plsc

_LANES = 16
_DIM = 64
_CODES = 16
_UNROLL = 8

_mesh = plsc.VectorSubcoreMesh(
    core_axis_name="c", subcore_axis_name="s", num_cores=1, num_subcores=1
)


@functools.partial(
    pl.kernel,
    out_type=(
        jax.ShapeDtypeStruct((1, _CODES), jnp.float32),
        jax.ShapeDtypeStruct((1, 1, _DIM), jnp.float32),
    ),
    mesh=_mesh,
    compiler_params=pltpu.CompilerParams(
        needs_layout_passes=False,
        disable_bounds_checks=True,
    ),
    scratch_types=[
        pltpu.VMEM((1, _DIM), jnp.float32),
        pltpu.VMEM((1, _CODES, _DIM), jnp.float32),
        pltpu.VMEM((1, _CODES), jnp.float32),
        pltpu.VMEM((1, 1, _DIM), jnp.float32),
        pltpu.SemaphoreType.DMA,
        pltpu.SemaphoreType.DMA,
    ],
)
def _vq_kernel(x_hbm, cb_hbm, onehot_hbm, resid_hbm, x_v, cb_v, oh_v, r_v,
               sem_a, sem_b):
    @pl.when(lax.axis_index("s") == 0)
    def _():
        in_a = pltpu.async_copy(cb_hbm, cb_v, sem_a)
        in_b = pltpu.async_copy(x_hbm, x_v, sem_b)
        in_a.wait()
        in_b.wait()
        lanes = lax.iota(jnp.int32, _LANES)
        zero = jnp.zeros((_LANES,), jnp.int32)

        def dist_body(i, accs):
            base = i * _UNROLL
            out = list(accs)
            for k in range(_UNROLL):
                d_splat = jnp.broadcast_to(base + k, (_LANES,))
                col = plsc.load_gather(cb_v, [zero, lanes, d_splat])
                xb = plsc.load_gather(x_v, [zero, d_splat])
                t = xb - col
                out[k % 4] = out[k % 4] + t * t
            return tuple(out)

        z = jnp.zeros((_LANES,), jnp.float32)
        acc = lax.fori_loop(0, _DIM // _UNROLL, dist_body, (z, z, z, z))
        dist = (acc[0] + acc[1]) + (acc[2] + acc[3])
        m = jnp.min(dist)
        idx = plsc.all_reduce_ffs(dist == m)
        oh_v[0, :] = jnp.where(lanes == idx, 1.0, 0.0).astype(jnp.float32)
        out_a = pltpu.async_copy(oh_v, onehot_hbm, sem_a)
        for i in range(_DIM // _LANES):
            xi = x_v[0, pl.ds(_LANES * i, _LANES)]
            row = plsc.load_gather(cb_v, [zero, idx, lanes + _LANES * i])
            r_v[0, 0, pl.ds(_LANES * i, _LANES)] = xi - row
        out_b = pltpu.async_copy(r_v, resid_hbm, sem_b)
        out_a.wait()
        out_b.wait()


def kernel(inputs, codebook):
    return _vq_kernel(inputs, codebook)


# speedup vs baseline: 1.0046x; 1.0046x over previous
import functools

import jax
import jax.numpy as jnp
from jax import lax
from jax.experimental import pallas as pl
from jax.experimental.pallas import tpu as pltpu
from jax.experimental.pallas import tpu_sc as plsc

_LANES = 16
_DIM = 64
_CODES = 16
_UNROLL = 4

_mesh = plsc.VectorSubcoreMesh(
    core_axis_name="c", subcore_axis_name="s", num_cores=1, num_subcores=1
)


@functools.partial(
    pl.kernel,
    out_type=(
        jax.ShapeDtypeStruct((1, _CODES), jnp.float32),
        jax.ShapeDtypeStruct((1, 1, _DIM), jnp.float32),
    ),
    mesh=_mesh,
    compiler_params=pltpu.CompilerParams(
        needs_layout_passes=False,
        disable_bounds_checks=True,
    ),
    scratch_types=[
        pltpu.VMEM((1, _DIM), jnp.float32),
        pltpu.VMEM((1, _CODES, _DIM), jnp.float32),
        pltpu.VMEM((1, _CODES), jnp.float32),
        pltpu.VMEM((1, 1, _DIM), jnp.float32),
        pltpu.SemaphoreType.DMA,
        pltpu.SemaphoreType.DMA,
    ],
)
def _vq_kernel(x_hbm, cb_hbm, onehot_hbm, resid_hbm, x_v, cb_v, oh_v, r_v,
               sem_a, sem_b):
    @pl.when(lax.axis_index("s") == 0)
    def _():
        in_a = pltpu.async_copy(cb_hbm, cb_v, sem_a)
        in_b = pltpu.async_copy(x_hbm, x_v, sem_b)
        in_a.wait()
        in_b.wait()
        lanes = lax.iota(jnp.int32, _LANES)
        zero = jnp.zeros((_LANES,), jnp.int32)

        def dist_body(i, accs):
            base = i * _UNROLL
            out = []
            for k in range(_UNROLL):
                d_splat = jnp.broadcast_to(base + k, (_LANES,))
                col = plsc.load_gather(cb_v, [zero, lanes, d_splat])
                xb = plsc.load_gather(x_v, [zero, d_splat])
                t = xb - col
                out.append(accs[k] + t * t)
            return tuple(out)

        z = jnp.zeros((_LANES,), jnp.float32)
        acc = lax.fori_loop(0, _DIM // _UNROLL, dist_body, (z, z, z, z))
        dist = (acc[0] + acc[1]) + (acc[2] + acc[3])
        m = jnp.min(dist)
        idx = plsc.all_reduce_ffs(dist == m)
        oh_v[0, :] = jnp.where(lanes == idx, 1.0, 0.0).astype(jnp.float32)
        out_a = pltpu.async_copy(oh_v, onehot_hbm, sem_a)
        for i in range(_DIM // _LANES):
            xi = x_v[0, pl.ds(_LANES * i, _LANES)]
            row = plsc.load_gather(cb_v, [zero, idx, lanes + _LANES * i])
            r_v[0, 0, pl.ds(_LANES * i, _LANES)] = xi - row
        out_b = pltpu.async_copy(r_v, resid_hbm, sem_b)
        out_a.wait()
        out_b.wait()


def kernel(inputs, codebook):
    return _vq_kernel(inputs, codebook)


# speedup vs baseline: 1.0140x; 1.0094x over previous
import functools

import jax
import jax.numpy as jnp
from jax import lax
from jax.experimental import pallas as pl
from jax.experimental.pallas import tpu as pltpu
from jax.experimental.pallas import tpu_sc as plsc

_LANES = 16
_DIM = 64
_CODES = 16
_UNROLL = 4

_mesh = plsc.VectorSubcoreMesh(
    core_axis_name="c", subcore_axis_name="s", num_cores=1, num_subcores=1
)


@functools.partial(
    pl.kernel,
    out_type=(
        jax.ShapeDtypeStruct((1, _CODES), jnp.float32),
        jax.ShapeDtypeStruct((1, 1, _DIM), jnp.float32),
    ),
    mesh=_mesh,
    compiler_params=pltpu.CompilerParams(
        needs_layout_passes=False,
        disable_bounds_checks=True,
        use_tc_tiling_on_sc=False,
    ),
    scratch_types=[
        pltpu.VMEM((1, _DIM), jnp.float32),
        pltpu.VMEM((1, _CODES, _DIM), jnp.float32),
        pltpu.VMEM((1, _CODES), jnp.float32),
        pltpu.VMEM((1, 1, _DIM), jnp.float32),
        pltpu.SemaphoreType.DMA,
        pltpu.SemaphoreType.DMA,
    ],
)
def _vq_kernel(x_hbm, cb_hbm, onehot_hbm, resid_hbm, x_v, cb_v, oh_v, r_v,
               sem_a, sem_b):
    @pl.when(lax.axis_index("s") == 0)
    def _():
        in_a = pltpu.async_copy(cb_hbm, cb_v, sem_a)
        in_b = pltpu.async_copy(x_hbm, x_v, sem_b)
        in_a.wait()
        in_b.wait()
        lanes = lax.iota(jnp.int32, _LANES)
        zero = jnp.zeros((_LANES,), jnp.int32)

        def dist_body(i, accs):
            base = i * _UNROLL
            out = []
            for k in range(_UNROLL):
                d_splat = jnp.broadcast_to(base + k, (_LANES,))
                col = plsc.load_gather(cb_v, [zero, lanes, d_splat])
                xb = plsc.load_gather(x_v, [zero, d_splat])
                t = xb - col
                out.append(accs[k] + t * t)
            return tuple(out)

        z = jnp.zeros((_LANES,), jnp.float32)
        acc = lax.fori_loop(0, _DIM // _UNROLL, dist_body, (z, z, z, z))
        dist = (acc[0] + acc[1]) + (acc[2] + acc[3])
        m = jnp.min(dist)
        idx = plsc.all_reduce_ffs(dist == m)
        oh_v[0, :] = jnp.where(lanes == idx, 1.0, 0.0).astype(jnp.float32)
        out_a = pltpu.async_copy(oh_v, onehot_hbm, sem_a)
        for i in range(_DIM // _LANES):
            xi = x_v[0, pl.ds(_LANES * i, _LANES)]
            row = plsc.load_gather(cb_v, [zero, idx, lanes + _LANES * i])
            r_v[0, 0, pl.ds(_LANES * i, _LANES)] = xi - row
        out_b = pltpu.async_copy(r_v, resid_hbm, sem_b)
        out_a.wait()
        out_b.wait()


def kernel(inputs, codebook):
    return _vq_kernel(inputs, codebook)


# speedup vs baseline: 1.0229x; 1.0088x over previous
import functools

import jax
import jax.numpy as jnp
from jax import lax
from jax.experimental import pallas as pl
from jax.experimental.pallas import tpu as pltpu
from jax.experimental.pallas import tpu_sc as plsc

_LANES = 16
_DIM = 64
_CODES = 16
_UNROLL = 4

_mesh = plsc.VectorSubcoreMesh(
    core_axis_name="c", subcore_axis_name="s", num_cores=1, num_subcores=1
)


@functools.partial(
    pl.kernel,
    out_type=(
        jax.ShapeDtypeStruct((1, _CODES), jnp.float32),
        jax.ShapeDtypeStruct((1, 1, _DIM), jnp.float32),
    ),
    mesh=_mesh,
    compiler_params=pltpu.CompilerParams(
        needs_layout_passes=False,
        disable_bounds_checks=True,
        use_tc_tiling_on_sc=False,
    ),
    scratch_types=[
        pltpu.VMEM((_CODES + 1, _DIM), jnp.float32),
        pltpu.VMEM((1, _CODES), jnp.float32),
        pltpu.VMEM((1, 1, _DIM), jnp.float32),
        pltpu.SemaphoreType.DMA,
        pltpu.SemaphoreType.DMA,
    ],
)
def _vq_kernel(xin_hbm, onehot_hbm, resid_hbm, in_v, oh_v, r_v, sem_a, sem_b):
    @pl.when(lax.axis_index("s") == 0)
    def _():
        pltpu.async_copy(xin_hbm, in_v, sem_a).wait()
        lanes = lax.iota(jnp.int32, _LANES)
        zero = jnp.zeros((_LANES,), jnp.int32)
        code_rows = lanes + 1

        def dist_body(i, accs):
            base = i * _UNROLL
            out = []
            for k in range(_UNROLL):
                d_splat = jnp.broadcast_to(base + k, (_LANES,))
                col = plsc.load_gather(in_v, [code_rows, d_splat])
                xb = plsc.load_gather(in_v, [zero, d_splat])
                t = xb - col
                out.append(accs[k] + t * t)
            return tuple(out)

        z = jnp.zeros((_LANES,), jnp.float32)
        acc = lax.fori_loop(0, _DIM // _UNROLL, dist_body, (z, z, z, z))
        dist = (acc[0] + acc[1]) + (acc[2] + acc[3])
        m = jnp.min(dist)
        idx = plsc.all_reduce_ffs(dist == m)
        oh_v[0, :] = jnp.where(lanes == idx, 1.0, 0.0).astype(jnp.float32)
        out_a = pltpu.async_copy(oh_v, onehot_hbm, sem_a)
        for i in range(_DIM // _LANES):
            xi = in_v[0, pl.ds(_LANES * i, _LANES)]
            row = plsc.load_gather(in_v, [idx + 1, lanes + _LANES * i])
            r_v[0, 0, pl.ds(_LANES * i, _LANES)] = xi - row
        out_b = pltpu.async_copy(r_v, resid_hbm, sem_b)
        out_a.wait()
        out_b.wait()


def kernel(inputs, codebook):
    xin = jnp.concatenate(
        [jnp.reshape(inputs, (1, _DIM)), jnp.reshape(codebook, (_CODES, _DIM))],
        axis=0,
    )
    return _vq_kernel(xin)
